# baseline (speedup 1.0000x reference)
import functools

import jax
import jax.numpy as jnp
from jax import lax
from jax.experimental import pallas as pl
from jax.experimental.pallas import tpu as pltpu
from jax.experimental.pallas import tpu_sc as plsc

N_ENT = 10000
E = 320000
D = 128
NUM_REL = 101
B = 1024

NC = 2
NS = 16
L = 16
NW = NC * NS

EROW = 128
EPT = 10240
E_PAD = NW * EPT
PAD_E = E_PAD - E
RPT = EPT // EROW

N_PAD = 10240
ZROW = 128
NZ = N_PAD // ZROW // NS

_mesh = plsc.VectorSubcoreMesh(
    core_axis_name="c", subcore_axis_name="s", num_cores=NC, num_subcores=NS)


def _zero_rows(buf, nrows, ncols):
    def body(i, _):
        for j in range(ncols // L):
            buf[i, pl.ds(j * L, L)] = jnp.zeros((L,), jnp.float32)
        return 0
    lax.fori_loop(0, nrows, body, 0)


def _make_sc_pass(with_rel):
    nf = 3 if with_rel else 2

    scratch = [pltpu.VMEM((EROW,), jnp.int32)]
    if with_rel:
        scratch += [pltpu.VMEM((EROW,), jnp.int32)]
    scratch += [pltpu.VMEM((1, EROW), jnp.int32)]
    scratch += [pltpu.VMEM((EROW, D), jnp.float32)]
    if with_rel:
        scratch += [pltpu.VMEM((EROW, D), jnp.float32)]
    scratch += [pltpu.VMEM_SHARED((N_PAD, D), jnp.float32)]
    if with_rel:
        scratch += [pltpu.VMEM_SHARED((NUM_REL, D), jnp.float32)]
    scratch += [pltpu.SemaphoreType.DMA for _ in range(2 if with_rel else 1)]

    def body(*refs):
        if with_rel:
            (tab_hbm, rel_hbm, pidx_hbm, out_hbm, sidx, eidx, didx,
             srows, rrows, agg, rel_sh, se, sr) = refs
        else:
            (tab_hbm, pidx_hbm, out_hbm, sidx, didx,
             srows, agg, se) = refs

        c = lax.axis_index("c")
        s = lax.axis_index("s")
        wid = s * NC + c
        base = wid * RPT

        _zero_rows(srows, EROW, D)

        def zc(k, _):
            chunk = s + k * NS
            pltpu.sync_copy(srows, agg.at[pl.ds(chunk * ZROW, ZROW)])
            return 0
        lax.fori_loop(0, NZ, zc, 0)

        if with_rel:
            @pl.when(s == 0)
            def _():
                pltpu.sync_copy(rel_hbm, rel_sh)

        plsc.subcore_barrier()

        def mul():
            def mbody(i2, _):
                for v in range(2):
                    for jj in range(D // L):
                        r = 2 * i2 + v
                        sl = pl.ds(jj * L, L)
                        srows[r, sl] = srows[r, sl] * rrows[r, sl]
                return 0
            lax.fori_loop(0, EROW // 2, mbody, 0)

        def rowloop(i, _):
            r = base + i
            pltpu.sync_copy(pidx_hbm.at[r * nf], sidx)
            if with_rel:
                pltpu.sync_copy(pidx_hbm.at[r * nf + 1], eidx)
            pltpu.sync_copy(pidx_hbm.at[r * nf + nf - 1], didx.at[0])
            cp0 = pltpu.async_copy(tab_hbm.at[sidx], srows, se)
            if with_rel:
                cp1 = pltpu.async_copy(rel_sh.at[eidx], rrows, sr)
            cp0.wait()
            if with_rel:
                cp1.wait()
                mul()
            pltpu.sync_copy(srows, agg.at[didx.at[0]], add=True)
            return 0
        lax.fori_loop(0, RPT, rowloop, 0)

        plsc.subcore_barrier()

        def wb(k, _):
            chunk = s + k * NS
            sl = pl.ds(chunk * ZROW, ZROW)
            pltpu.sync_copy(agg.at[sl], out_hbm.at[c, sl])
            return 0
        lax.fori_loop(0, NZ, wb, 0)

    return pl.kernel(
        body,
        out_type=jax.ShapeDtypeStruct((NC, N_PAD, D), jnp.float32),
        mesh=_mesh,
        scratch_types=scratch,
    )


_sc_msg_pass = _make_sc_pass(with_rel=True)
_sc_agg_pass = _make_sc_pass(with_rel=False)


def _bn_relu(x, g, b):
    mu = jnp.mean(x, axis=0, keepdims=True)
    var = jnp.mean((x - mu) ** 2, axis=0, keepdims=True)
    return jnp.maximum((x - mu) / jnp.sqrt(var + 1e-5) * g + b, 0.0)


def _tc_proj_body(emb_h_ref, w_e_ref, b_e_ref, rel_wt_ref, emb_e_ref,
                  ent_out, rel_out):
    ent_out[...] = (
        jnp.dot(emb_h_ref[...], w_e_ref[...], preferred_element_type=jnp.float32)
        + b_e_ref[...]
    )
    rel_out[...] = jnp.dot(
        rel_wt_ref[...], emb_e_ref[...], preferred_element_type=jnp.float32
    )


def _tc_bn0_body(p_ref, ent_ref, relrow_ref, g_ref, b_ref, out_ref):
    agg = (p_ref[0, :N_ENT, :] + p_ref[1, :N_ENT, :]
           + ent_ref[...] * relrow_ref[...])
    out_ref[...] = _bn_relu(agg, g_ref[...], b_ref[...])


def _tc_head_body(p_ref, z_ref, rel_e_ref, w_rel_ref, subj_ref,
                  rel_ref, wtop_ref, wbot_ref, cb_ref, g1_ref, b1_ref,
                  gc_ref, bc_ref, h_out, q_out):
    z = z_ref[...]
    agg1 = p_ref[0, :N_ENT, :] + p_ref[1, :N_ENT, :] + z
    h1 = _bn_relu(agg1, g1_ref[...], b1_ref[...])
    hc = (
        jnp.dot(z, wtop_ref[...], preferred_element_type=jnp.float32)
        + jnp.dot(h1, wbot_ref[...], preferred_element_type=jnp.float32)
        + cb_ref[...]
    )
    h = _bn_relu(hc, gc_ref[...], bc_ref[...])
    h_out[...] = h

    rel2 = jnp.dot(rel_e_ref[...], w_rel_ref[...], preferred_element_type=jnp.float32)
    ohr = (rel_ref[...] == lax.broadcasted_iota(jnp.int32, (B, NUM_REL), 1))
    q_r = jnp.dot(ohr.astype(jnp.float32), rel2, preferred_element_type=jnp.float32)

    subj = subj_ref[...]
    acc = jnp.zeros((B, D), jnp.float32)
    blk = 2000
    for k in range(N_ENT // blk):
        iota = lax.broadcasted_iota(jnp.int32, (B, blk), 1) + k * blk
        oh = (subj == iota).astype(jnp.float32)
        acc = acc + jnp.dot(oh, h[k * blk:(k + 1) * blk, :],
                            preferred_element_type=jnp.float32)
    q_out[...] = acc * q_r


def _tc_score_body(q_ref, h_ref, out_ref):
    out_ref[...] = lax.dot_general(
        q_ref[...], h_ref[...],
        (((1,), (1,)), ((), ())),
        preferred_element_type=jnp.float32,
    )


def kernel(edge_index, edge_type, subj, rel, emb_h, emb_e, W_e, b_e, rel_wt,
           w_rel, bn0_g, bn0_b, bn1_g, bn1_b, concat_W, concat_b, bnc_g, bnc_b):
    i32 = jnp.int32
    src_f = jnp.concatenate([edge_index[0].astype(i32), jnp.zeros((PAD_E,), i32)])
    dst_f = jnp.concatenate([edge_index[1].astype(i32), jnp.full((PAD_E,), N_ENT, i32)])
    et_f = jnp.concatenate([edge_type.astype(i32), jnp.zeros((PAD_E,), i32)])

    def _pack(arrs):
        parts = [a.reshape(NW * RPT, 1, EROW) for a in arrs]
        return jnp.concatenate(parts, axis=1).reshape(-1, EROW)

    pidx1 = _pack([src_f, et_f, dst_f])
    pidx2 = _pack([src_f, dst_f])
    subj2d = subj.astype(i32).reshape(B, 1)
    rel2d = rel.astype(i32).reshape(B, 1)

    ent, rel_embed = pl.pallas_call(
        _tc_proj_body,
        out_shape=(
            jax.ShapeDtypeStruct((N_ENT, D), jnp.float32),
            jax.ShapeDtypeStruct((NUM_REL, D), jnp.float32),
        ),
    )(emb_h, W_e, b_e.reshape(1, D), rel_wt, emb_e)

    p1 = _sc_msg_pass(ent, rel_embed, pidx1)

    zero_out = pl.pallas_call(
        _tc_bn0_body,
        out_shape=jax.ShapeDtypeStruct((N_ENT, D), jnp.float32),
    )(p1, ent, rel_embed[NUM_REL - 1:NUM_REL], bn0_g.reshape(1, D),
      bn0_b.reshape(1, D))

    p2 = _sc_agg_pass(zero_out, pidx2)

    h, q = pl.pallas_call(
        _tc_head_body,
        out_shape=(
            jax.ShapeDtypeStruct((N_ENT, D), jnp.float32),
            jax.ShapeDtypeStruct((B, D), jnp.float32),
        ),
    )(p2, zero_out, rel_embed, w_rel, subj2d, rel2d,
      concat_W[:D], concat_W[D:], concat_b.reshape(1, D),
      bn1_g.reshape(1, D), bn1_b.reshape(1, D),
      bnc_g.reshape(1, D), bnc_b.reshape(1, D))

    score = pl.pallas_call(
        _tc_score_body,
        out_shape=jax.ShapeDtypeStruct((B, N_ENT), jnp.float32),
    )(q, h)
    return score


# speedup vs baseline: 1.9572x; 1.9572x over previous
import functools

import jax
import jax.numpy as jnp
from jax import lax
from jax.experimental import pallas as pl
from jax.experimental.pallas import tpu as pltpu
from jax.experimental.pallas import tpu_sc as plsc

N_ENT = 10000
E = 320000
D = 128
NUM_REL = 101
B = 1024

NC = 2
NS = 16
L = 16
NW = NC * NS

EROW = 128
NROWS = E // EROW
N_PAD = 10240
NCHUNK = N_PAD // EROW

_mesh = plsc.VectorSubcoreMesh(
    core_axis_name="c", subcore_axis_name="s", num_cores=NC, num_subcores=NS)


def _zero_rows(buf, nrows, ncols):
    def body(i, _):
        for j in range(ncols // L):
            buf[i, pl.ds(j * L, L)] = jnp.zeros((L,), jnp.float32)
        return 0
    lax.fori_loop(0, nrows, body, 0)


@functools.partial(
    pl.kernel,
    out_type=jax.ShapeDtypeStruct((NC, N_PAD, D), jnp.float32),
    mesh=_mesh,
    scratch_types=[
        pltpu.VMEM((EROW,), jnp.int32),
        pltpu.VMEM((EROW,), jnp.int32),
        pltpu.VMEM((1, EROW), jnp.int32),
        pltpu.VMEM((EROW, D), jnp.float32),
        pltpu.VMEM((EROW, D), jnp.float32),
        pltpu.VMEM_SHARED((N_PAD, D), jnp.float32),
        pltpu.VMEM_SHARED((NUM_REL, D), jnp.float32),
        pltpu.SemaphoreType.DMA,
        pltpu.SemaphoreType.DMA,
    ],
)
def _sc_pass1(ent_hbm, rel_hbm, src_hbm, et_hbm, dst_hbm, out_hbm,
              sidx, eidx, didx, srows, rrows, agg, rel_sh, sem0, sem1):
    c = lax.axis_index("c")
    s = lax.axis_index("s")
    wid = s * NC + c

    _zero_rows(srows, EROW, D)

    def zc(k, _):
        chunk = s + k * NS
        pltpu.sync_copy(srows, agg.at[pl.ds(chunk * EROW, EROW)])
        return 0
    lax.fori_loop(0, NCHUNK // NS, zc, 0)

    @pl.when(s == 0)
    def _():
        pltpu.sync_copy(rel_hbm, rel_sh)

    plsc.subcore_barrier()

    nmine = (NROWS - wid + NW - 1) // NW

    def body(i, _):
        r = wid + i * NW
        pltpu.sync_copy(src_hbm.at[r], sidx)
        pltpu.sync_copy(et_hbm.at[r], eidx)
        pltpu.sync_copy(dst_hbm.at[r], didx.at[0])
        cp0 = pltpu.async_copy(ent_hbm.at[sidx], srows, sem0)
        cp1 = pltpu.async_copy(rel_sh.at[eidx], rrows, sem1)
        cp0.wait()
        cp1.wait()

        def mul(i2, _):
            for j in range(D // L):
                sl = pl.ds(j * L, L)
                srows[i2, sl] = srows[i2, sl] * rrows[i2, sl]
            return 0
        lax.fori_loop(0, EROW, mul, 0)

        pltpu.sync_copy(srows, agg.at[didx.at[0]], add=True)
        return 0
    lax.fori_loop(0, nmine, body, 0)

    plsc.subcore_barrier()

    def wb(k, _):
        chunk = s + k * NS
        sl = pl.ds(chunk * EROW, EROW)
        pltpu.sync_copy(agg.at[sl], out_hbm.at[c, sl])
        return 0
    lax.fori_loop(0, NCHUNK // NS, wb, 0)


@functools.partial(
    pl.kernel,
    out_type=jax.ShapeDtypeStruct((NC, N_PAD, D), jnp.float32),
    mesh=_mesh,
    scratch_types=[
        pltpu.VMEM((EROW,), jnp.int32),
        pltpu.VMEM((1, EROW), jnp.int32),
        pltpu.VMEM((EROW, D), jnp.float32),
        pltpu.VMEM_SHARED((N_PAD, D), jnp.float32),
        pltpu.SemaphoreType.DMA,
    ],
)
def _sc_pass2(node_hbm, src_hbm, dst_hbm, out_hbm,
              sidx, didx, srows, agg, sem0):
    c = lax.axis_index("c")
    s = lax.axis_index("s")
    wid = s * NC + c

    _zero_rows(srows, EROW, D)

    def zc(k, _):
        chunk = s + k * NS
        pltpu.sync_copy(srows, agg.at[pl.ds(chunk * EROW, EROW)])
        return 0
    lax.fori_loop(0, NCHUNK // NS, zc, 0)

    plsc.subcore_barrier()

    nmine = (NROWS - wid + NW - 1) // NW

    def body(i, _):
        r = wid + i * NW
        pltpu.sync_copy(src_hbm.at[r], sidx)
        pltpu.sync_copy(dst_hbm.at[r], didx.at[0])
        pltpu.async_copy(node_hbm.at[sidx], srows, sem0).wait()
        pltpu.sync_copy(srows, agg.at[didx.at[0]], add=True)
        return 0
    lax.fori_loop(0, nmine, body, 0)

    plsc.subcore_barrier()

    def wb(k, _):
        chunk = s + k * NS
        sl = pl.ds(chunk * EROW, EROW)
        pltpu.sync_copy(agg.at[sl], out_hbm.at[c, sl])
        return 0
    lax.fori_loop(0, NCHUNK // NS, wb, 0)


def _bn_relu(x, g, b):
    mu = jnp.mean(x, axis=0, keepdims=True)
    var = jnp.mean((x - mu) ** 2, axis=0, keepdims=True)
    return jnp.maximum((x - mu) / jnp.sqrt(var + 1e-5) * g + b, 0.0)


def _tc_proj_body(emb_h_ref, w_e_ref, b_e_ref, rel_wt_ref, emb_e_ref,
                  ent_out, rel_out):
    ent_out[...] = (
        jnp.dot(emb_h_ref[...], w_e_ref[...], preferred_element_type=jnp.float32)
        + b_e_ref[...]
    )
    rel_out[...] = jnp.dot(
        rel_wt_ref[...], emb_e_ref[...], preferred_element_type=jnp.float32
    )


def _tc_bn0_body(p_ref, ent_ref, relrow_ref, g_ref, b_ref, out_ref):
    agg = (p_ref[0, :N_ENT, :] + p_ref[1, :N_ENT, :]
           + ent_ref[...] * relrow_ref[...])
    out_ref[...] = _bn_relu(agg, g_ref[...], b_ref[...])


def _tc_head_body(p_ref, z_ref, rel_e_ref, w_rel_ref, subj_ref,
                  rel_ref, wtop_ref, wbot_ref, cb_ref, g1_ref, b1_ref,
                  gc_ref, bc_ref, h_out, q_out):
    z = z_ref[...]
    agg1 = p_ref[0, :N_ENT, :] + p_ref[1, :N_ENT, :] + z
    h1 = _bn_relu(agg1, g1_ref[...], b1_ref[...])
    hc = (
        jnp.dot(z, wtop_ref[...], preferred_element_type=jnp.float32)
        + jnp.dot(h1, wbot_ref[...], preferred_element_type=jnp.float32)
        + cb_ref[...]
    )
    h = _bn_relu(hc, gc_ref[...], bc_ref[...])
    h_out[...] = h

    rel2 = jnp.dot(rel_e_ref[...], w_rel_ref[...], preferred_element_type=jnp.float32)
    ohr = (rel_ref[...] == lax.broadcasted_iota(jnp.int32, (B, NUM_REL), 1))
    q_r = jnp.dot(ohr.astype(jnp.float32), rel2, preferred_element_type=jnp.float32)

    subj = subj_ref[...]
    acc = jnp.zeros((B, D), jnp.float32)
    blk = 2000
    for k in range(N_ENT // blk):
        iota = lax.broadcasted_iota(jnp.int32, (B, blk), 1) + k * blk
        oh = (subj == iota).astype(jnp.float32)
        acc = acc + jnp.dot(oh, h[k * blk:(k + 1) * blk, :],
                            preferred_element_type=jnp.float32)
    q_out[...] = acc * q_r


def _tc_score_body(q_ref, h_ref, out_ref):
    out_ref[...] = lax.dot_general(
        q_ref[...], h_ref[...],
        (((1,), (1,)), ((), ())),
        preferred_element_type=jnp.float32,
    )


def kernel(edge_index, edge_type, subj, rel, emb_h, emb_e, W_e, b_e, rel_wt,
           w_rel, bn0_g, bn0_b, bn1_g, bn1_b, concat_W, concat_b, bnc_g, bnc_b):
    i32 = jnp.int32
    src2d = edge_index[0].astype(i32).reshape(NROWS, EROW)
    dst2d = edge_index[1].astype(i32).reshape(NROWS, EROW)
    et2d = edge_type.astype(i32).reshape(NROWS, EROW)
    subj2d = subj.astype(i32).reshape(B, 1)
    rel2d = rel.astype(i32).reshape(B, 1)

    ent, rel_embed = pl.pallas_call(
        _tc_proj_body,
        out_shape=(
            jax.ShapeDtypeStruct((N_ENT, D), jnp.float32),
            jax.ShapeDtypeStruct((NUM_REL, D), jnp.float32),
        ),
    )(emb_h, W_e, b_e.reshape(1, D), rel_wt, emb_e)

    p1 = _sc_pass1(ent, rel_embed, src2d, et2d, dst2d)

    zero_out = pl.pallas_call(
        _tc_bn0_body,
        out_shape=jax.ShapeDtypeStruct((N_ENT, D), jnp.float32),
    )(p1, ent, rel_embed[NUM_REL - 1:NUM_REL], bn0_g.reshape(1, D),
      bn0_b.reshape(1, D))

    p2 = _sc_pass2(zero_out, src2d, dst2d)

    h, q = pl.pallas_call(
        _tc_head_body,
        out_shape=(
            jax.ShapeDtypeStruct((N_ENT, D), jnp.float32),
            jax.ShapeDtypeStruct((B, D), jnp.float32),
        ),
    )(p2, zero_out, rel_embed, w_rel, subj2d, rel2d,
      concat_W[:D], concat_W[D:], concat_b.reshape(1, D),
      bn1_g.reshape(1, D), bn1_b.reshape(1, D),
      bnc_g.reshape(1, D), bnc_b.reshape(1, D))

    score = pl.pallas_call(
        _tc_score_body,
        out_shape=jax.ShapeDtypeStruct((B, N_ENT), jnp.float32),
    )(q, h)
    return score


# speedup vs baseline: 2.3861x; 1.2191x over previous
import functools

import jax
import jax.numpy as jnp
from jax import lax
from jax.experimental import pallas as pl
from jax.experimental.pallas import tpu as pltpu
from jax.experimental.pallas import tpu_sc as plsc

N_ENT = 10000
E = 320000
D = 128
NUM_REL = 101
B = 1024

NC = 2
NS = 16
L = 16
NW = NC * NS

EROW = 128
NROWS = E // EROW
N_PAD = 10240
NCHUNK = N_PAD // EROW

_mesh = plsc.VectorSubcoreMesh(
    core_axis_name="c", subcore_axis_name="s", num_cores=NC, num_subcores=NS)


def _zero_rows(buf, nrows, ncols):
    def body(i, _):
        for j in range(ncols // L):
            buf[i, pl.ds(j * L, L)] = jnp.zeros((L,), jnp.float32)
        return 0
    lax.fori_loop(0, nrows, body, 0)


@functools.partial(
    pl.kernel,
    out_type=jax.ShapeDtypeStruct((NC, N_PAD, D), jnp.float32),
    mesh=_mesh,
    scratch_types=[
        pltpu.VMEM((EROW,), jnp.int32),
        pltpu.VMEM((EROW,), jnp.int32),
        pltpu.VMEM((1, EROW), jnp.int32),
        pltpu.VMEM((EROW, D), jnp.float32),
        pltpu.VMEM((EROW, D), jnp.float32),
        pltpu.VMEM_SHARED((N_PAD, D), jnp.float32),
        pltpu.VMEM_SHARED((NUM_REL, D), jnp.float32),
        pltpu.SemaphoreType.DMA,
        pltpu.SemaphoreType.DMA,
        pltpu.SemaphoreType.DMA,
        pltpu.SemaphoreType.DMA,
        pltpu.SemaphoreType.DMA,
    ],
)
def _sc_pass1(ent_hbm, rel_hbm, src_hbm, et_hbm, dst_hbm, out_hbm,
              sidx, eidx, didx, srows, rrows, agg, rel_sh,
              sem0, sem1, sia, sib, sic):
    c = lax.axis_index("c")
    s = lax.axis_index("s")
    wid = s * NC + c

    _zero_rows(srows, EROW, D)

    def zc(k, _):
        chunk = s + k * NS
        pltpu.sync_copy(srows, agg.at[pl.ds(chunk * EROW, EROW)])
        return 0
    lax.fori_loop(0, NCHUNK // NS, zc, 0)

    @pl.when(s == 0)
    def _():
        pltpu.sync_copy(rel_hbm, rel_sh)

    plsc.subcore_barrier()

    nmine = (NROWS - wid + NW - 1) // NW

    def body(i, _):
        r = wid + i * NW
        ca = pltpu.async_copy(src_hbm.at[r], sidx, sia)
        cb = pltpu.async_copy(et_hbm.at[r], eidx, sib)
        cc = pltpu.async_copy(dst_hbm.at[r], didx.at[0], sic)
        ca.wait()
        cp0 = pltpu.async_copy(ent_hbm.at[sidx], srows, sem0)
        cb.wait()
        cp1 = pltpu.async_copy(rel_sh.at[eidx], rrows, sem1)
        cp0.wait()
        cp1.wait()

        def mul(i2, _):
            for v in range(2):
                for j in range(D // L):
                    rr = 2 * i2 + v
                    sl = pl.ds(j * L, L)
                    srows[rr, sl] = srows[rr, sl] * rrows[rr, sl]
            return 0
        lax.fori_loop(0, EROW // 2, mul, 0)

        cc.wait()
        pltpu.sync_copy(srows, agg.at[didx.at[0]], add=True)
        return 0
    lax.fori_loop(0, nmine, body, 0)

    plsc.subcore_barrier()

    def wb(k, _):
        chunk = s + k * NS
        sl = pl.ds(chunk * EROW, EROW)
        pltpu.sync_copy(agg.at[sl], out_hbm.at[c, sl])
        return 0
    lax.fori_loop(0, NCHUNK // NS, wb, 0)


@functools.partial(
    pl.kernel,
    out_type=jax.ShapeDtypeStruct((NC, N_PAD, D), jnp.float32),
    mesh=_mesh,
    scratch_types=[
        pltpu.VMEM((EROW,), jnp.int32),
        pltpu.VMEM((1, EROW), jnp.int32),
        pltpu.VMEM((EROW, D), jnp.float32),
        pltpu.VMEM_SHARED((N_PAD, D), jnp.float32),
        pltpu.SemaphoreType.DMA,
        pltpu.SemaphoreType.DMA,
        pltpu.SemaphoreType.DMA,
    ],
)
def _sc_pass2(node_hbm, src_hbm, dst_hbm, out_hbm,
              sidx, didx, srows, agg, sem0, sia, sic):
    c = lax.axis_index("c")
    s = lax.axis_index("s")
    wid = s * NC + c

    _zero_rows(srows, EROW, D)

    def zc(k, _):
        chunk = s + k * NS
        pltpu.sync_copy(srows, agg.at[pl.ds(chunk * EROW, EROW)])
        return 0
    lax.fori_loop(0, NCHUNK // NS, zc, 0)

    plsc.subcore_barrier()

    nmine = (NROWS - wid + NW - 1) // NW

    def body(i, _):
        r = wid + i * NW
        ca = pltpu.async_copy(src_hbm.at[r], sidx, sia)
        cc = pltpu.async_copy(dst_hbm.at[r], didx.at[0], sic)
        ca.wait()
        pltpu.async_copy(node_hbm.at[sidx], srows, sem0).wait()
        cc.wait()
        pltpu.sync_copy(srows, agg.at[didx.at[0]], add=True)
        return 0
    lax.fori_loop(0, nmine, body, 0)

    plsc.subcore_barrier()

    def wb(k, _):
        chunk = s + k * NS
        sl = pl.ds(chunk * EROW, EROW)
        pltpu.sync_copy(agg.at[sl], out_hbm.at[c, sl])
        return 0
    lax.fori_loop(0, NCHUNK // NS, wb, 0)


def _bn_relu(x, g, b):
    mu = jnp.mean(x, axis=0, keepdims=True)
    var = jnp.mean((x - mu) ** 2, axis=0, keepdims=True)
    return jnp.maximum((x - mu) / jnp.sqrt(var + 1e-5) * g + b, 0.0)


def _tc_proj_body(emb_h_ref, w_e_ref, b_e_ref, rel_wt_ref, emb_e_ref,
                  ent_out, rel_out):
    ent_out[...] = (
        jnp.dot(emb_h_ref[...], w_e_ref[...], preferred_element_type=jnp.float32)
        + b_e_ref[...]
    )
    rel_out[...] = jnp.dot(
        rel_wt_ref[...], emb_e_ref[...], preferred_element_type=jnp.float32
    )


def _tc_bn0_body(p_ref, ent_ref, relrow_ref, g_ref, b_ref, out_ref):
    agg = (p_ref[0, :N_ENT, :] + p_ref[1, :N_ENT, :]
           + ent_ref[...] * relrow_ref[...])
    out_ref[...] = _bn_relu(agg, g_ref[...], b_ref[...])


def _tc_head_body(p_ref, z_ref, rel_e_ref, w_rel_ref, subj_ref,
                  rel_ref, wtop_ref, wbot_ref, cb_ref, g1_ref, b1_ref,
                  gc_ref, bc_ref, h_out, q_out):
    z = z_ref[...]
    agg1 = p_ref[0, :N_ENT, :] + p_ref[1, :N_ENT, :] + z
    h1 = _bn_relu(agg1, g1_ref[...], b1_ref[...])
    hc = (
        jnp.dot(z, wtop_ref[...], preferred_element_type=jnp.float32)
        + jnp.dot(h1, wbot_ref[...], preferred_element_type=jnp.float32)
        + cb_ref[...]
    )
    h = _bn_relu(hc, gc_ref[...], bc_ref[...])
    h_out[...] = h

    rel2 = jnp.dot(rel_e_ref[...], w_rel_ref[...], preferred_element_type=jnp.float32)
    ohr = (rel_ref[...] == lax.broadcasted_iota(jnp.int32, (B, NUM_REL), 1))
    q_r = jnp.dot(ohr.astype(jnp.float32), rel2, preferred_element_type=jnp.float32)

    subj = subj_ref[...]
    acc = jnp.zeros((B, D), jnp.float32)
    blk = 2000
    for k in range(N_ENT // blk):
        iota = lax.broadcasted_iota(jnp.int32, (B, blk), 1) + k * blk
        oh = (subj == iota).astype(jnp.float32)
        acc = acc + jnp.dot(oh, h[k * blk:(k + 1) * blk, :],
                            preferred_element_type=jnp.float32)
    q_out[...] = acc * q_r


def _tc_score_body(q_ref, h_ref, out_ref):
    out_ref[...] = lax.dot_general(
        q_ref[...], h_ref[...],
        (((1,), (1,)), ((), ())),
        preferred_element_type=jnp.float32,
    )


def kernel(edge_index, edge_type, subj, rel, emb_h, emb_e, W_e, b_e, rel_wt,
           w_rel, bn0_g, bn0_b, bn1_g, bn1_b, concat_W, concat_b, bnc_g, bnc_b):
    i32 = jnp.int32
    src2d = edge_index[0].astype(i32).reshape(NROWS, EROW)
    dst2d = edge_index[1].astype(i32).reshape(NROWS, EROW)
    et2d = edge_type.astype(i32).reshape(NROWS, EROW)
    subj2d = subj.astype(i32).reshape(B, 1)
    rel2d = rel.astype(i32).reshape(B, 1)

    ent, rel_embed = pl.pallas_call(
        _tc_proj_body,
        out_shape=(
            jax.ShapeDtypeStruct((N_ENT, D), jnp.float32),
            jax.ShapeDtypeStruct((NUM_REL, D), jnp.float32),
        ),
    )(emb_h, W_e, b_e.reshape(1, D), rel_wt, emb_e)

    p1 = _sc_pass1(ent, rel_embed, src2d, et2d, dst2d)

    zero_out = pl.pallas_call(
        _tc_bn0_body,
        out_shape=jax.ShapeDtypeStruct((N_ENT, D), jnp.float32),
    )(p1, ent, rel_embed[NUM_REL - 1:NUM_REL], bn0_g.reshape(1, D),
      bn0_b.reshape(1, D))

    p2 = _sc_pass2(zero_out, src2d, dst2d)

    h, q = pl.pallas_call(
        _tc_head_body,
        out_shape=(
            jax.ShapeDtypeStruct((N_ENT, D), jnp.float32),
            jax.ShapeDtypeStruct((B, D), jnp.float32),
        ),
    )(p2, zero_out, rel_embed, w_rel, subj2d, rel2d,
      concat_W[:D], concat_W[D:], concat_b.reshape(1, D),
      bn1_g.reshape(1, D), bn1_b.reshape(1, D),
      bnc_g.reshape(1, D), bnc_b.reshape(1, D))

    score = pl.pallas_call(
        _tc_score_body,
        out_shape=jax.ShapeDtypeStruct((B, N_ENT), jnp.float32),
    )(q, h)
    return score


# speedup vs baseline: 2.7556x; 1.1548x over previous
import functools

import jax
import jax.numpy as jnp
from jax import lax
from jax.experimental import pallas as pl
from jax.experimental.pallas import tpu as pltpu
from jax.experimental.pallas import tpu_sc as plsc

N_ENT = 10000
E = 320000
D = 128
NUM_REL = 101
B = 1024

NC = 2
NS = 16
L = 16
NW = NC * NS

EROW = 128
NROWS = E // EROW
N_PAD = 10240
NCHUNK = N_PAD // EROW

_mesh = plsc.VectorSubcoreMesh(
    core_axis_name="c", subcore_axis_name="s", num_cores=NC, num_subcores=NS)


def _zero_rows(buf, nrows, ncols):
    def body(i, _):
        for j in range(ncols // L):
            buf[i, pl.ds(j * L, L)] = jnp.zeros((L,), jnp.float32)
        return 0
    lax.fori_loop(0, nrows, body, 0)


@functools.partial(
    pl.kernel,
    out_type=jax.ShapeDtypeStruct((NC, N_PAD, D), jnp.float32),
    mesh=_mesh,
    scratch_types=[
        pltpu.VMEM((EROW,), jnp.int32),
        pltpu.VMEM((EROW,), jnp.int32),
        pltpu.VMEM((1, EROW), jnp.int32),
        pltpu.VMEM((EROW, D), jnp.float32),
        pltpu.VMEM((EROW, D), jnp.float32),
        pltpu.VMEM_SHARED((N_PAD, D), jnp.float32),
        pltpu.VMEM_SHARED((NUM_REL, D), jnp.float32),
        pltpu.SemaphoreType.DMA,
        pltpu.SemaphoreType.DMA,
        pltpu.SemaphoreType.DMA,
        pltpu.SemaphoreType.DMA,
        pltpu.SemaphoreType.DMA,
    ],
)
def _sc_pass1(ent_hbm, rel_hbm, src_hbm, et_hbm, dst_hbm, out_hbm,
              sidx, eidx, didx, srows, rrows, agg, rel_sh,
              sem0, sem1, sia, sib, sic):
    c = lax.axis_index("c")
    s = lax.axis_index("s")
    wid = s * NC + c

    _zero_rows(srows, EROW, D)

    def zc(k, _):
        chunk = s + k * NS
        pltpu.sync_copy(srows, agg.at[pl.ds(chunk * EROW, EROW)])
        return 0
    lax.fori_loop(0, NCHUNK // NS, zc, 0)

    @pl.when(s == 0)
    def _():
        pltpu.sync_copy(rel_hbm, rel_sh)

    plsc.subcore_barrier()

    nmine = (NROWS - wid + NW - 1) // NW

    def body(i, _):
        r = wid + i * NW
        ca = pltpu.async_copy(src_hbm.at[r], sidx, sia)
        cb = pltpu.async_copy(et_hbm.at[r], eidx, sib)
        cc = pltpu.async_copy(dst_hbm.at[r], didx.at[0], sic)
        ca.wait()
        cp0 = pltpu.async_copy(ent_hbm.at[sidx], srows, sem0)
        cb.wait()
        cp1 = pltpu.async_copy(rel_sh.at[eidx], rrows, sem1)
        cp0.wait()
        cp1.wait()

        def mul(i2, _):
            for v in range(2):
                for j in range(D // L):
                    rr = 2 * i2 + v
                    sl = pl.ds(j * L, L)
                    srows[rr, sl] = srows[rr, sl] * rrows[rr, sl]
            return 0
        lax.fori_loop(0, EROW // 2, mul, 0)

        cc.wait()
        pltpu.sync_copy(srows, agg.at[didx.at[0]], add=True)
        return 0
    lax.fori_loop(0, nmine, body, 0)

    plsc.subcore_barrier()

    def wb(k, _):
        chunk = s + k * NS
        sl = pl.ds(chunk * EROW, EROW)
        pltpu.sync_copy(agg.at[sl], out_hbm.at[c, sl])
        return 0
    lax.fori_loop(0, NCHUNK // NS, wb, 0)


@functools.partial(
    pl.kernel,
    out_type=jax.ShapeDtypeStruct((NC, N_PAD, D), jnp.float32),
    mesh=_mesh,
    scratch_types=[
        pltpu.VMEM((EROW,), jnp.int32),
        pltpu.VMEM((EROW,), jnp.int32),
        pltpu.VMEM((1, EROW), jnp.int32),
        pltpu.VMEM((1, EROW), jnp.int32),
        pltpu.VMEM((EROW, D), jnp.float32),
        pltpu.VMEM((EROW, D), jnp.float32),
        pltpu.VMEM_SHARED((N_PAD, D), jnp.float32),
        pltpu.SemaphoreType.DMA,
        pltpu.SemaphoreType.DMA,
        pltpu.SemaphoreType.DMA,
        pltpu.SemaphoreType.DMA,
        pltpu.SemaphoreType.DMA,
        pltpu.SemaphoreType.DMA,
    ],
)
def _sc_pass2(node_hbm, src_hbm, dst_hbm, out_hbm,
              si0, si1, di0, di1, sr0, sr1, agg,
              ge0, ge1, sa0, sa1, sc0, sc1):
    c = lax.axis_index("c")
    s = lax.axis_index("s")
    wid = s * NC + c

    sidx = (si0, si1)
    didx = (di0, di1)
    srows = (sr0, sr1)
    ge = (ge0, ge1)
    sa = (sa0, sa1)
    sc_ = (sc0, sc1)

    _zero_rows(sr0, EROW, D)

    def zc(k, _):
        chunk = s + k * NS
        pltpu.sync_copy(sr0, agg.at[pl.ds(chunk * EROW, EROW)])
        return 0
    lax.fori_loop(0, NCHUNK // NS, zc, 0)

    plsc.subcore_barrier()

    nmine = (NROWS - wid + NW - 1) // NW

    def rrow(i):
        return wid + i * NW

    def fire_idx(i, d):
        pltpu.async_copy(src_hbm.at[rrow(i)], sidx[d], sa[d])
        pltpu.async_copy(dst_hbm.at[rrow(i)], didx[d].at[0], sc_[d])

    def wait_idx(i, d):
        pltpu.make_async_copy(src_hbm.at[rrow(i)], sidx[d], sa[d]).wait()
        pltpu.make_async_copy(dst_hbm.at[rrow(i)], didx[d].at[0], sc_[d]).wait()

    def fire_g(d):
        pltpu.async_copy(node_hbm.at[sidx[d]], srows[d], ge[d])

    def wait_g(d):
        pltpu.make_async_copy(node_hbm.at[sidx[d]], srows[d], ge[d]).wait()

    def halfrow(i, d):
        @pl.when(i + 1 < nmine)
        def _():
            wait_idx(i + 1, 1 - d)
            fire_g(1 - d)
        wait_g(d)
        pltpu.sync_copy(srows[d], agg.at[didx[d].at[0]], add=True)

        @pl.when(i + 2 < nmine)
        def _():
            fire_idx(i + 2, d)

    pltpu.sync_copy(src_hbm.at[rrow(0)], si0)
    pltpu.sync_copy(dst_hbm.at[rrow(0)], di0.at[0])
    fire_g(0)

    @pl.when(nmine > 1)
    def _():
        fire_idx(1, 1)

    def pair(k, _):
        halfrow(2 * k, 0)
        halfrow(2 * k + 1, 1)
        return 0
    lax.fori_loop(0, nmine // 2, pair, 0)

    @pl.when(nmine % 2 == 1)
    def _():
        i = nmine - 1
        wait_g(0)
        pltpu.sync_copy(srows[0], agg.at[didx[0].at[0]], add=True)

    plsc.subcore_barrier()

    def wb(k, _):
        chunk = s + k * NS
        sl = pl.ds(chunk * EROW, EROW)
        pltpu.sync_copy(agg.at[sl], out_hbm.at[c, sl])
        return 0
    lax.fori_loop(0, NCHUNK // NS, wb, 0)


def _bn_relu(x, g, b):
    mu = jnp.mean(x, axis=0, keepdims=True)
    var = jnp.mean((x - mu) ** 2, axis=0, keepdims=True)
    return jnp.maximum((x - mu) / jnp.sqrt(var + 1e-5) * g + b, 0.0)


def _tc_proj_body(emb_h_ref, w_e_ref, b_e_ref, rel_wt_ref, emb_e_ref,
                  ent_out, rel_out):
    ent_out[...] = (
        jnp.dot(emb_h_ref[...], w_e_ref[...], preferred_element_type=jnp.float32)
        + b_e_ref[...]
    )
    rel_out[...] = jnp.dot(
        rel_wt_ref[...], emb_e_ref[...], preferred_element_type=jnp.float32
    )


def _tc_bn0_body(p_ref, ent_ref, relrow_ref, g_ref, b_ref, out_ref):
    agg = (p_ref[0, :N_ENT, :] + p_ref[1, :N_ENT, :]
           + ent_ref[...] * relrow_ref[...])
    out_ref[...] = _bn_relu(agg, g_ref[...], b_ref[...])


def _tc_head_body(p_ref, z_ref, rel_e_ref, w_rel_ref, subj_ref,
                  rel_ref, wtop_ref, wbot_ref, cb_ref, g1_ref, b1_ref,
                  gc_ref, bc_ref, h_out, q_out):
    z = z_ref[...]
    agg1 = p_ref[0, :N_ENT, :] + p_ref[1, :N_ENT, :] + z
    h1 = _bn_relu(agg1, g1_ref[...], b1_ref[...])
    hc = (
        jnp.dot(z, wtop_ref[...], preferred_element_type=jnp.float32)
        + jnp.dot(h1, wbot_ref[...], preferred_element_type=jnp.float32)
        + cb_ref[...]
    )
    h = _bn_relu(hc, gc_ref[...], bc_ref[...])
    h_out[...] = h

    rel2 = jnp.dot(rel_e_ref[...], w_rel_ref[...], preferred_element_type=jnp.float32)
    ohr = (rel_ref[...] == lax.broadcasted_iota(jnp.int32, (B, NUM_REL), 1))
    q_r = jnp.dot(ohr.astype(jnp.float32), rel2, preferred_element_type=jnp.float32)

    subj = subj_ref[...]
    acc = jnp.zeros((B, D), jnp.float32)
    blk = 2000
    for k in range(N_ENT // blk):
        iota = lax.broadcasted_iota(jnp.int32, (B, blk), 1) + k * blk
        oh = (subj == iota).astype(jnp.float32)
        acc = acc + jnp.dot(oh, h[k * blk:(k + 1) * blk, :],
                            preferred_element_type=jnp.float32)
    q_out[...] = acc * q_r


def _tc_score_body(q_ref, h_ref, out_ref):
    out_ref[...] = lax.dot_general(
        q_ref[...], h_ref[...],
        (((1,), (1,)), ((), ())),
        preferred_element_type=jnp.float32,
    )


def kernel(edge_index, edge_type, subj, rel, emb_h, emb_e, W_e, b_e, rel_wt,
           w_rel, bn0_g, bn0_b, bn1_g, bn1_b, concat_W, concat_b, bnc_g, bnc_b):
    i32 = jnp.int32
    src2d = edge_index[0].astype(i32).reshape(NROWS, EROW)
    dst2d = edge_index[1].astype(i32).reshape(NROWS, EROW)
    et2d = edge_type.astype(i32).reshape(NROWS, EROW)
    subj2d = subj.astype(i32).reshape(B, 1)
    rel2d = rel.astype(i32).reshape(B, 1)

    ent, rel_embed = pl.pallas_call(
        _tc_proj_body,
        out_shape=(
            jax.ShapeDtypeStruct((N_ENT, D), jnp.float32),
            jax.ShapeDtypeStruct((NUM_REL, D), jnp.float32),
        ),
    )(emb_h, W_e, b_e.reshape(1, D), rel_wt, emb_e)

    p1 = _sc_pass1(ent, rel_embed, src2d, et2d, dst2d)

    zero_out = pl.pallas_call(
        _tc_bn0_body,
        out_shape=jax.ShapeDtypeStruct((N_ENT, D), jnp.float32),
    )(p1, ent, rel_embed[NUM_REL - 1:NUM_REL], bn0_g.reshape(1, D),
      bn0_b.reshape(1, D))

    p2 = _sc_pass2(zero_out, src2d, dst2d)

    h, q = pl.pallas_call(
        _tc_head_body,
        out_shape=(
            jax.ShapeDtypeStruct((N_ENT, D), jnp.float32),
            jax.ShapeDtypeStruct((B, D), jnp.float32),
        ),
    )(p2, zero_out, rel_embed, w_rel, subj2d, rel2d,
      concat_W[:D], concat_W[D:], concat_b.reshape(1, D),
      bn1_g.reshape(1, D), bn1_b.reshape(1, D),
      bnc_g.reshape(1, D), bnc_b.reshape(1, D))

    score = pl.pallas_call(
        _tc_score_body,
        out_shape=jax.ShapeDtypeStruct((B, N_ENT), jnp.float32),
    )(q, h)
    return score


# speedup vs baseline: 3.0720x; 1.1148x over previous
import functools

import jax
import jax.numpy as jnp
from jax import lax
from jax.experimental import pallas as pl
from jax.experimental.pallas import tpu as pltpu
from jax.experimental.pallas import tpu_sc as plsc

N_ENT = 10000
E = 320000
D = 128
NUM_REL = 101
B = 1024

NC = 2
NS = 16
L = 16
NW = NC * NS

EROW = 128
NROWS = E // EROW
EROW1 = 64
NROWS1 = E // EROW1
N_PAD = 10240
NCHUNK = N_PAD // EROW

_mesh = plsc.VectorSubcoreMesh(
    core_axis_name="c", subcore_axis_name="s", num_cores=NC, num_subcores=NS)


def _zero_rows(buf, nrows, ncols):
    def body(i, _):
        for j in range(ncols // L):
            buf[i, pl.ds(j * L, L)] = jnp.zeros((L,), jnp.float32)
        return 0
    lax.fori_loop(0, nrows, body, 0)


@functools.partial(
    pl.kernel,
    out_type=jax.ShapeDtypeStruct((NC, N_PAD, D), jnp.float32),
    mesh=_mesh,
    scratch_types=[
        pltpu.VMEM((EROW1,), jnp.int32),
        pltpu.VMEM((EROW1,), jnp.int32),
        pltpu.VMEM((EROW1,), jnp.int32),
        pltpu.VMEM((EROW1,), jnp.int32),
        pltpu.VMEM((1, EROW1), jnp.int32),
        pltpu.VMEM((1, EROW1), jnp.int32),
        pltpu.VMEM((EROW1, D), jnp.float32),
        pltpu.VMEM((EROW1, D), jnp.float32),
        pltpu.VMEM((EROW1, D), jnp.float32),
        pltpu.VMEM((EROW1, D), jnp.float32),
        pltpu.VMEM_SHARED((N_PAD, D), jnp.float32),
        pltpu.VMEM_SHARED((NUM_REL, D), jnp.float32),
        pltpu.SemaphoreType.DMA,
        pltpu.SemaphoreType.DMA,
        pltpu.SemaphoreType.DMA,
        pltpu.SemaphoreType.DMA,
        pltpu.SemaphoreType.DMA,
        pltpu.SemaphoreType.DMA,
        pltpu.SemaphoreType.DMA,
        pltpu.SemaphoreType.DMA,
        pltpu.SemaphoreType.DMA,
        pltpu.SemaphoreType.DMA,
    ],
)
def _sc_pass1(ent_hbm, rel_hbm, src_hbm, et_hbm, dst_hbm, out_hbm,
              si0, si1, ei0, ei1, di0, di1, sr0, sr1, rr0, rr1,
              agg, rel_sh,
              ge0, ge1, gr0, gr1, sa0, sa1, sb0, sb1, sc0, sc1):
    c = lax.axis_index("c")
    s = lax.axis_index("s")
    wid = s * NC + c

    sidx = (si0, si1)
    eidx = (ei0, ei1)
    didx = (di0, di1)
    srows = (sr0, sr1)
    rrows = (rr0, rr1)
    ge = (ge0, ge1)
    gr = (gr0, gr1)
    sa = (sa0, sa1)
    sb = (sb0, sb1)
    sc_ = (sc0, sc1)

    _zero_rows(sr0, EROW1, D)

    def zc(k, _):
        chunk = s + k * NS
        pltpu.sync_copy(sr0, agg.at[pl.ds(chunk * EROW1, EROW1)])
        return 0
    lax.fori_loop(0, N_PAD // EROW1 // NS, zc, 0)

    @pl.when(s == 0)
    def _():
        pltpu.sync_copy(rel_hbm, rel_sh)

    plsc.subcore_barrier()

    nmine = (NROWS1 - wid + NW - 1) // NW

    def rrow(i):
        return wid + i * NW

    def fire_idx(i, d):
        pltpu.async_copy(src_hbm.at[rrow(i)], sidx[d], sa[d])
        pltpu.async_copy(et_hbm.at[rrow(i)], eidx[d], sb[d])
        pltpu.async_copy(dst_hbm.at[rrow(i)], didx[d].at[0], sc_[d])

    def wait_idx(i, d):
        pltpu.make_async_copy(src_hbm.at[rrow(i)], sidx[d], sa[d]).wait()
        pltpu.make_async_copy(et_hbm.at[rrow(i)], eidx[d], sb[d]).wait()
        pltpu.make_async_copy(dst_hbm.at[rrow(i)], didx[d].at[0], sc_[d]).wait()

    def fire_g(d):
        pltpu.async_copy(ent_hbm.at[sidx[d]], srows[d], ge[d])
        pltpu.async_copy(rel_sh.at[eidx[d]], rrows[d], gr[d])

    def wait_g(d):
        pltpu.make_async_copy(ent_hbm.at[sidx[d]], srows[d], ge[d]).wait()
        pltpu.make_async_copy(rel_sh.at[eidx[d]], rrows[d], gr[d]).wait()

    def mul_scatter(d):
        def mul(i2, _):
            for v in range(2):
                for j in range(D // L):
                    rr = 2 * i2 + v
                    sl = pl.ds(j * L, L)
                    srows[d][rr, sl] = srows[d][rr, sl] * rrows[d][rr, sl]
            return 0
        lax.fori_loop(0, EROW1 // 2, mul, 0)
        pltpu.sync_copy(srows[d], agg.at[didx[d].at[0]], add=True)

    def halfrow(i, d):
        @pl.when(i + 1 < nmine)
        def _():
            wait_idx(i + 1, 1 - d)
            fire_g(1 - d)
        wait_g(d)
        mul_scatter(d)

        @pl.when(i + 2 < nmine)
        def _():
            fire_idx(i + 2, d)

    pltpu.sync_copy(src_hbm.at[rrow(0)], si0)
    pltpu.sync_copy(et_hbm.at[rrow(0)], ei0)
    pltpu.sync_copy(dst_hbm.at[rrow(0)], di0.at[0])
    fire_g(0)

    @pl.when(nmine > 1)
    def _():
        fire_idx(1, 1)

    def pair(k, _):
        halfrow(2 * k, 0)
        halfrow(2 * k + 1, 1)
        return 0
    lax.fori_loop(0, nmine // 2, pair, 0)

    @pl.when(nmine % 2 == 1)
    def _():
        wait_g(0)
        mul_scatter(0)

    plsc.subcore_barrier()

    def wb(k, _):
        chunk = s + k * NS
        sl = pl.ds(chunk * EROW, EROW)
        pltpu.sync_copy(agg.at[sl], out_hbm.at[c, sl])
        return 0
    lax.fori_loop(0, NCHUNK // NS, wb, 0)


@functools.partial(
    pl.kernel,
    out_type=jax.ShapeDtypeStruct((NC, N_PAD, D), jnp.float32),
    mesh=_mesh,
    scratch_types=[
        pltpu.VMEM((EROW,), jnp.int32),
        pltpu.VMEM((EROW,), jnp.int32),
        pltpu.VMEM((1, EROW), jnp.int32),
        pltpu.VMEM((1, EROW), jnp.int32),
        pltpu.VMEM((EROW, D), jnp.float32),
        pltpu.VMEM((EROW, D), jnp.float32),
        pltpu.VMEM_SHARED((N_PAD, D), jnp.float32),
        pltpu.SemaphoreType.DMA,
        pltpu.SemaphoreType.DMA,
        pltpu.SemaphoreType.DMA,
        pltpu.SemaphoreType.DMA,
        pltpu.SemaphoreType.DMA,
        pltpu.SemaphoreType.DMA,
    ],
)
def _sc_pass2(node_hbm, src_hbm, dst_hbm, out_hbm,
              si0, si1, di0, di1, sr0, sr1, agg,
              ge0, ge1, sa0, sa1, sc0, sc1):
    c = lax.axis_index("c")
    s = lax.axis_index("s")
    wid = s * NC + c

    sidx = (si0, si1)
    didx = (di0, di1)
    srows = (sr0, sr1)
    ge = (ge0, ge1)
    sa = (sa0, sa1)
    sc_ = (sc0, sc1)

    _zero_rows(sr0, EROW, D)

    def zc(k, _):
        chunk = s + k * NS
        pltpu.sync_copy(sr0, agg.at[pl.ds(chunk * EROW, EROW)])
        return 0
    lax.fori_loop(0, NCHUNK // NS, zc, 0)

    plsc.subcore_barrier()

    nmine = (NROWS - wid + NW - 1) // NW

    def rrow(i):
        return wid + i * NW

    def fire_idx(i, d):
        pltpu.async_copy(src_hbm.at[rrow(i)], sidx[d], sa[d])
        pltpu.async_copy(dst_hbm.at[rrow(i)], didx[d].at[0], sc_[d])

    def wait_idx(i, d):
        pltpu.make_async_copy(src_hbm.at[rrow(i)], sidx[d], sa[d]).wait()
        pltpu.make_async_copy(dst_hbm.at[rrow(i)], didx[d].at[0], sc_[d]).wait()

    def fire_g(d):
        pltpu.async_copy(node_hbm.at[sidx[d]], srows[d], ge[d])

    def wait_g(d):
        pltpu.make_async_copy(node_hbm.at[sidx[d]], srows[d], ge[d]).wait()

    def halfrow(i, d):
        @pl.when(i + 1 < nmine)
        def _():
            wait_idx(i + 1, 1 - d)
            fire_g(1 - d)
        wait_g(d)
        pltpu.sync_copy(srows[d], agg.at[didx[d].at[0]], add=True)

        @pl.when(i + 2 < nmine)
        def _():
            fire_idx(i + 2, d)

    pltpu.sync_copy(src_hbm.at[rrow(0)], si0)
    pltpu.sync_copy(dst_hbm.at[rrow(0)], di0.at[0])
    fire_g(0)

    @pl.when(nmine > 1)
    def _():
        fire_idx(1, 1)

    def pair(k, _):
        halfrow(2 * k, 0)
        halfrow(2 * k + 1, 1)
        return 0
    lax.fori_loop(0, nmine // 2, pair, 0)

    @pl.when(nmine % 2 == 1)
    def _():
        i = nmine - 1
        wait_g(0)
        pltpu.sync_copy(srows[0], agg.at[didx[0].at[0]], add=True)

    plsc.subcore_barrier()

    def wb(k, _):
        chunk = s + k * NS
        sl = pl.ds(chunk * EROW, EROW)
        pltpu.sync_copy(agg.at[sl], out_hbm.at[c, sl])
        return 0
    lax.fori_loop(0, NCHUNK // NS, wb, 0)


def _bn_relu(x, g, b):
    mu = jnp.mean(x, axis=0, keepdims=True)
    var = jnp.mean((x - mu) ** 2, axis=0, keepdims=True)
    return jnp.maximum((x - mu) / jnp.sqrt(var + 1e-5) * g + b, 0.0)


def _tc_proj_body(emb_h_ref, w_e_ref, b_e_ref, rel_wt_ref, emb_e_ref,
                  ent_out, rel_out):
    ent_out[...] = (
        jnp.dot(emb_h_ref[...], w_e_ref[...], preferred_element_type=jnp.float32)
        + b_e_ref[...]
    )
    rel_out[...] = jnp.dot(
        rel_wt_ref[...], emb_e_ref[...], preferred_element_type=jnp.float32
    )


def _tc_bn0_body(p_ref, ent_ref, relrow_ref, g_ref, b_ref, out_ref):
    agg = (p_ref[0, :N_ENT, :] + p_ref[1, :N_ENT, :]
           + ent_ref[...] * relrow_ref[...])
    out_ref[...] = _bn_relu(agg, g_ref[...], b_ref[...])


def _tc_head_body(p_ref, z_ref, rel_e_ref, w_rel_ref, subj_ref,
                  rel_ref, wtop_ref, wbot_ref, cb_ref, g1_ref, b1_ref,
                  gc_ref, bc_ref, h_out, q_out):
    z = z_ref[...]
    agg1 = p_ref[0, :N_ENT, :] + p_ref[1, :N_ENT, :] + z
    h1 = _bn_relu(agg1, g1_ref[...], b1_ref[...])
    hc = (
        jnp.dot(z, wtop_ref[...], preferred_element_type=jnp.float32)
        + jnp.dot(h1, wbot_ref[...], preferred_element_type=jnp.float32)
        + cb_ref[...]
    )
    h = _bn_relu(hc, gc_ref[...], bc_ref[...])
    h_out[...] = h

    rel2 = jnp.dot(rel_e_ref[...], w_rel_ref[...], preferred_element_type=jnp.float32)
    ohr = (rel_ref[...] == lax.broadcasted_iota(jnp.int32, (B, NUM_REL), 1))
    q_r = jnp.dot(ohr.astype(jnp.float32), rel2, preferred_element_type=jnp.float32)

    subj = subj_ref[...]
    acc = jnp.zeros((B, D), jnp.float32)
    blk = 2000
    for k in range(N_ENT // blk):
        iota = lax.broadcasted_iota(jnp.int32, (B, blk), 1) + k * blk
        oh = (subj == iota).astype(jnp.float32)
        acc = acc + jnp.dot(oh, h[k * blk:(k + 1) * blk, :],
                            preferred_element_type=jnp.float32)
    q_out[...] = acc * q_r


def _tc_score_body(q_ref, h_ref, out_ref):
    out_ref[...] = lax.dot_general(
        q_ref[...], h_ref[...],
        (((1,), (1,)), ((), ())),
        preferred_element_type=jnp.float32,
    )


def kernel(edge_index, edge_type, subj, rel, emb_h, emb_e, W_e, b_e, rel_wt,
           w_rel, bn0_g, bn0_b, bn1_g, bn1_b, concat_W, concat_b, bnc_g, bnc_b):
    i32 = jnp.int32
    src2d = edge_index[0].astype(i32).reshape(NROWS, EROW)
    dst2d = edge_index[1].astype(i32).reshape(NROWS, EROW)
    src64 = edge_index[0].astype(i32).reshape(NROWS1, EROW1)
    dst64 = edge_index[1].astype(i32).reshape(NROWS1, EROW1)
    et64 = edge_type.astype(i32).reshape(NROWS1, EROW1)
    subj2d = subj.astype(i32).reshape(B, 1)
    rel2d = rel.astype(i32).reshape(B, 1)

    ent, rel_embed = pl.pallas_call(
        _tc_proj_body,
        out_shape=(
            jax.ShapeDtypeStruct((N_ENT, D), jnp.float32),
            jax.ShapeDtypeStruct((NUM_REL, D), jnp.float32),
        ),
    )(emb_h, W_e, b_e.reshape(1, D), rel_wt, emb_e)

    p1 = _sc_pass1(ent, rel_embed, src64, et64, dst64)

    zero_out = pl.pallas_call(
        _tc_bn0_body,
        out_shape=jax.ShapeDtypeStruct((N_ENT, D), jnp.float32),
    )(p1, ent, rel_embed[NUM_REL - 1:NUM_REL], bn0_g.reshape(1, D),
      bn0_b.reshape(1, D))

    p2 = _sc_pass2(zero_out, src2d, dst2d)

    h, q = pl.pallas_call(
        _tc_head_body,
        out_shape=(
            jax.ShapeDtypeStruct((N_ENT, D), jnp.float32),
            jax.ShapeDtypeStruct((B, D), jnp.float32),
        ),
    )(p2, zero_out, rel_embed, w_rel, subj2d, rel2d,
      concat_W[:D], concat_W[D:], concat_b.reshape(1, D),
      bn1_g.reshape(1, D), bn1_b.reshape(1, D),
      bnc_g.reshape(1, D), bnc_b.reshape(1, D))

    score = pl.pallas_call(
        _tc_score_body,
        out_shape=jax.ShapeDtypeStruct((B, N_ENT), jnp.float32),
    )(q, h)
    return score
